# baseline (speedup 1.0000x reference)
import jax
import jax.numpy as jnp
from jax import lax
from jax.experimental import pallas as pl
from jax.experimental.pallas import tpu as pltpu
from jax.experimental.pallas import tpu_sc as plsc

_NC, _NS = 2, 16
_NW = _NC * _NS

_RUNS_H = (
    ((1, 0, 11), (13, 11, 13), (27, 24, 2), (31, 26, 14)),
    ((46, 0, 20), (67, 20, 1), (70, 21, 1), (72, 22, 11), (84, 33, 7)),
)


def _sc_body(pred_in, enc_in, aux_in, pred_out, enc_out, aux_out,
             buf_a, buf_b, buf_t, si, so):
    wid = lax.axis_index("s") * _NC + lax.axis_index("c")

    def issue_ins(six, runs, b8, q0, qlen, buf):
        return [pltpu.async_copy(six(s, ln, b8, q0, qlen),
                                 buf.at[pl.ds(d, ln)], si)
                for s, d, ln in runs]

    def issue_outs(dix, cbase, b8, q0, qlen, buf):
        return [pltpu.async_copy(buf.at[:, b, :],
                                 dix(b8 + b, cbase, q0, qlen), so)
                for b in range(8)]

    def do_combo(six, dix, bg, q0):
        b8 = bg * 8
        ins = issue_ins(six, _RUNS_H[0], b8, q0, 128, buf_a)
        for dsc in ins:
            dsc.wait()
        ins = issue_ins(six, _RUNS_H[1], b8, q0, 128, buf_b)
        outs = issue_outs(dix, 0, b8, q0, 128, buf_a)
        for dsc in outs:
            dsc.wait()
        for dsc in ins:
            dsc.wait()
        outs = issue_outs(dix, 40, b8, q0, 128, buf_b)
        for dsc in outs:
            dsc.wait()

    def do_tail(six, dix, bg):
        b8 = bg * 8
        for h, cbase in ((0, 0), (1, 40)):
            ins = issue_ins(six, _RUNS_H[h], b8, 896, 4, buf_t)
            for dsc in ins:
                dsc.wait()
            outs = issue_outs(dix, cbase, b8, 896, 4, buf_t)
            for dsc in outs:
                dsc.wait()

    def make_ix(src, dst, a=None):
        if a is None:
            six = lambda s, ln, b8, q0, ql: src.at[
                pl.ds(s, ln), pl.ds(b8, 8), pl.ds(q0, ql)]
            dix = lambda bb, cb, q0, ql: dst.at[
                bb, pl.ds(cb, 40), pl.ds(q0, ql)]
        else:
            six = lambda s, ln, b8, q0, ql: src.at[
                a, pl.ds(s, ln), pl.ds(b8, 8), pl.ds(q0, ql)]
            dix = lambda bb, cb, q0, ql: dst.at[
                a, bb, pl.ds(cb, 40), pl.ds(q0, ql)]
        return six, dix

    def region(first_w, n_w, n_combos, src, dst, is_aux):
        @pl.when(jnp.logical_and(wid >= first_w, wid < first_w + n_w))
        def _():
            u = wid - first_w
            n_mine = (n_combos - u + n_w - 1) // n_w

            def combo_body(j, carry):
                c = u + j * n_w
                if is_aux:
                    a = c // 14
                    r = c % 14
                    six, dix = make_ix(src, dst, a)
                else:
                    r = c
                    six, dix = make_ix(src, dst)
                do_combo(six, dix, r // 7, (r % 7) * 128)
                return carry

            lax.fori_loop(0, n_mine, combo_body, 0)

    region(0, 4, 14, pred_in, pred_out, False)
    region(4, 4, 14, enc_in, enc_out, False)
    region(8, 24, 84, aux_in, aux_out, True)

    def tail_region(first_w, n_w, src, dst, is_aux):
        @pl.when(jnp.logical_and(wid >= first_w, wid < first_w + n_w))
        def _():
            u = wid - first_w
            if is_aux:
                six, dix = make_ix(src, dst, u // 2)
                bg = u % 2
            else:
                six, dix = make_ix(src, dst)
                bg = u
            do_tail(six, dix, bg)

    tail_region(0, 2, pred_in, pred_out, False)
    tail_region(4, 2, enc_in, enc_out, False)
    tail_region(8, 12, aux_in, aux_out, True)


@jax.jit
def kernel(pred_logits, enc_pred_logits, aux_pred_logits):
    mesh = plsc.VectorSubcoreMesh(core_axis_name="c", subcore_axis_name="s",
                                  num_cores=_NC, num_subcores=_NS)
    run = pl.kernel(
        _sc_body,
        out_type=(
            jax.ShapeDtypeStruct((16, 80, 900), jnp.float32),
            jax.ShapeDtypeStruct((16, 80, 900), jnp.float32),
            jax.ShapeDtypeStruct((6, 16, 80, 900), jnp.float32),
        ),
        mesh=mesh,
        scratch_types=[
            pltpu.VMEM((40, 8, 128), jnp.float32),
            pltpu.VMEM((40, 8, 128), jnp.float32),
            pltpu.VMEM((40, 8, 4), jnp.float32),
            pltpu.SemaphoreType.DMA,
            pltpu.SemaphoreType.DMA,
        ],
        compiler_params=pltpu.CompilerParams(needs_layout_passes=False),
    )
    out_t, enc_t, aux_t = run(pred_logits.transpose(2, 0, 1),
                              enc_pred_logits.transpose(2, 0, 1),
                              aux_pred_logits.transpose(0, 3, 1, 2))
    return (out_t.transpose(0, 2, 1),
            enc_t.transpose(0, 2, 1),
            aux_t.transpose(0, 1, 3, 2))


# speedup vs baseline: 1.0505x; 1.0505x over previous
import jax
import jax.numpy as jnp
from jax import lax
from jax.experimental import pallas as pl
from jax.experimental.pallas import tpu as pltpu
from jax.experimental.pallas import tpu_sc as plsc

_NC, _NS = 2, 16
_NW = _NC * _NS

_RUNS_H = (
    ((1, 0, 11), (13, 11, 13), (27, 24, 2), (31, 26, 14)),
    ((46, 0, 20), (67, 20, 1), (70, 21, 1), (72, 22, 11), (84, 33, 7)),
)


def _sc_body(pred_in, enc_in, aux_in, pred_out, enc_out, aux_out,
             buf_a, buf_b, si, so):
    wid = lax.axis_index("s") * _NC + lax.axis_index("c")

    def do_unit(six, dix, bg, q0, runs, buf, qlen, cbase):
        b8 = bg * 8
        for s, d, ln in runs:
            pltpu.async_copy(six(s, ln, b8, q0, qlen),
                             buf.at[pl.ds(d, ln)], si)
        for s, d, ln in runs:
            pltpu.make_async_copy(six(s, ln, b8, q0, qlen),
                                  buf.at[pl.ds(d, ln)], si).wait()
        for b in range(8):
            pltpu.async_copy(buf.at[:, b, :], dix(b8 + b, cbase, q0, qlen),
                             so)
        for b in range(8):
            pltpu.make_async_copy(buf.at[:, b, :],
                                  dix(b8 + b, cbase, q0, qlen), so).wait()

    def make_ix(src, dst, a=None):
        if a is None:
            six = lambda s, ln, b8, q0, ql: src.at[
                pl.ds(s, ln), pl.ds(b8, 8), pl.ds(q0, ql)]
            dix = lambda bb, cb, q0, ql: dst.at[
                bb, pl.ds(cb, 40), pl.ds(q0, ql)]
        else:
            six = lambda s, ln, b8, q0, ql: src.at[
                a, pl.ds(s, ln), pl.ds(b8, 8), pl.ds(q0, ql)]
            dix = lambda bb, cb, q0, ql: dst.at[
                a, bb, pl.ds(cb, 40), pl.ds(q0, ql)]
        return six, dix

    def dispatch_slab(slab, fn):
        def on_pred(_):
            fn(*make_ix(pred_in, pred_out))
            return 0

        def on_enc(_):
            fn(*make_ix(enc_in, enc_out))
            return 0

        def on_aux(_):
            fn(*make_ix(aux_in, aux_out, slab - 2))
            return 0

        return lax.switch(jnp.minimum(slab, 2), (on_pred, on_enc, on_aux), 0)

    def do_halves(six, dix, bg, q0, buf, qlen, ch):
        def h0(_):
            do_unit(six, dix, bg, q0, _RUNS_H[0], buf, qlen, 0)
            return 0

        def h1(_):
            do_unit(six, dix, bg, q0, _RUNS_H[1], buf, qlen, 40)
            return 0

        return lax.cond(ch == 0, h0, h1, 0)

    def unit_body(i, carry):
        g = wid + i * _NW

        def main(gg):
            slab = gg // 24
            r = gg % 24
            bg = r // 12
            qt = (r % 12) // 2
            ch = r % 2
            return dispatch_slab(
                slab,
                lambda six, dix: do_halves(six, dix, bg, qt * 128, buf_a,
                                           128, ch))

        def last(gg):
            v = gg - 192
            slab = v // 4
            bg = (v % 4) // 2
            ch = v % 2
            return dispatch_slab(
                slab,
                lambda six, dix: do_halves(six, dix, bg, 768, buf_b, 132,
                                           ch))

        lax.cond(g < 192, main, last, g)
        return carry

    lax.fori_loop(0, 7, unit_body, 0)


@jax.jit
def kernel(pred_logits, enc_pred_logits, aux_pred_logits):
    mesh = plsc.VectorSubcoreMesh(core_axis_name="c", subcore_axis_name="s",
                                  num_cores=_NC, num_subcores=_NS)
    run = pl.kernel(
        _sc_body,
        out_type=(
            jax.ShapeDtypeStruct((16, 80, 900), jnp.float32),
            jax.ShapeDtypeStruct((16, 80, 900), jnp.float32),
            jax.ShapeDtypeStruct((6, 16, 80, 900), jnp.float32),
        ),
        mesh=mesh,
        scratch_types=[
            pltpu.VMEM((40, 8, 128), jnp.float32),
            pltpu.VMEM((40, 8, 132), jnp.float32),
            pltpu.SemaphoreType.DMA,
            pltpu.SemaphoreType.DMA,
        ],
        compiler_params=pltpu.CompilerParams(needs_layout_passes=False),
    )
    out_t, enc_t, aux_t = run(pred_logits.transpose(2, 0, 1),
                              enc_pred_logits.transpose(2, 0, 1),
                              aux_pred_logits.transpose(0, 3, 1, 2))
    return (out_t.transpose(0, 2, 1),
            enc_t.transpose(0, 2, 1),
            aux_t.transpose(0, 1, 3, 2))
